# baseline (speedup 1.0000x reference)
import functools

import jax
import jax.numpy as jnp
from jax import lax
from jax.experimental import pallas as pl
from jax.experimental.pallas import tpu as pltpu
from jax.experimental.pallas import tpu_sc as plsc

N_NODES = 10000
N_EDGES = 320000
H = 128

NC = 2
NS = 16
NW = NC * NS
EPW = N_EDGES // NW

SUB = 80
NB = 1
GRP = SUB * NB
NGRP = EPW // GRP

SUB_S = 80
NGRP_S = EPW // SUB_S
NROWCH = N_NODES // SUB_S


def _premix_body(v_ref, ws_ref, wr_ref, ps_ref, pr_ref):
    v = v_ref[...]
    ps_ref[...] = jnp.dot(v, ws_ref[...], preferred_element_type=jnp.float32)
    pr_ref[...] = jnp.dot(v, wr_ref[...], preferred_element_type=jnp.float32)


def _premix(v, ws, wr):
    return pl.pallas_call(
        _premix_body,
        out_shape=(
            jax.ShapeDtypeStruct((N_NODES, H), jnp.float32),
            jax.ShapeDtypeStruct((N_NODES, H), jnp.float32),
        ),
    )(v, ws, wr)


def _edgein_body(e_ref, we_ref, b1_ref, o_ref):
    o_ref[...] = (jnp.dot(e_ref[...], we_ref[...],
                          preferred_element_type=jnp.float32) + b1_ref[...])


def _edgein(e, we, b1):
    bm = 512
    return pl.pallas_call(
        _edgein_body,
        grid=(N_EDGES // bm,),
        in_specs=[
            pl.BlockSpec((bm, H), lambda i: (i, 0)),
            pl.BlockSpec((H, H), lambda i: (0, 0)),
            pl.BlockSpec((1, H), lambda i: (0, 0)),
        ],
        out_specs=pl.BlockSpec((bm, H), lambda i: (i, 0)),
        out_shape=jax.ShapeDtypeStruct((N_EDGES, H), jnp.float32),
    )(e, we, b1.reshape(1, H))


def _combine_body(g_ref, e1_ref, w2_ref, b2_ref, o_ref):
    h = jnp.maximum(g_ref[...] + e1_ref[...], 0.0)
    o_ref[...] = jnp.dot(h, w2_ref[...], preferred_element_type=jnp.float32) + b2_ref[...]


def _combine(g, e1, w2, b2):
    bm = 512
    return pl.pallas_call(
        _combine_body,
        grid=(N_EDGES // bm,),
        in_specs=[
            pl.BlockSpec((bm, H), lambda i: (i, 0)),
            pl.BlockSpec((bm, H), lambda i: (i, 0)),
            pl.BlockSpec((H, H), lambda i: (0, 0)),
            pl.BlockSpec((1, H), lambda i: (0, 0)),
        ],
        out_specs=pl.BlockSpec((bm, H), lambda i: (i, 0)),
        out_shape=jax.ShapeDtypeStruct((N_EDGES, H), jnp.float32),
    )(g, e1, w2, b2.reshape(1, H))


def _vertex_body(v_ref, p_ref, wv_ref, wa_ref, b1_ref, w2_ref, b2_ref, o_ref):
    aggr = p_ref[0] + p_ref[1]
    pre = (jnp.dot(v_ref[...], wv_ref[...], preferred_element_type=jnp.float32)
           + jnp.dot(aggr, wa_ref[...], preferred_element_type=jnp.float32)
           + b1_ref[...])
    h = jnp.maximum(pre, 0.0)
    o_ref[...] = jnp.dot(h, w2_ref[...], preferred_element_type=jnp.float32) + b2_ref[...]


def _vertex_mlp(v, partials, wv, wa, b1, w2, b2):
    return pl.pallas_call(
        _vertex_body,
        out_shape=jax.ShapeDtypeStruct((N_NODES, H), jnp.float32),
    )(v, partials, wv, wa, b1.reshape(1, H), w2, b2.reshape(1, H))


def _gather_add(ps, pr, sidx, ridx):
    mesh = plsc.VectorSubcoreMesh(core_axis_name="c", subcore_axis_name="s")

    @functools.partial(
        pl.kernel,
        out_type=jax.ShapeDtypeStruct((N_EDGES, H), jnp.float32),
        mesh=mesh,
        scratch_types=[
            pltpu.VMEM((NGRP * NB, SUB), jnp.int32),
            pltpu.VMEM((NGRP * NB, SUB), jnp.int32),
            pltpu.VMEM((GRP, H), jnp.float32),
            pltpu.VMEM((GRP, H), jnp.float32),
            pltpu.VMEM((GRP, H), jnp.float32),
            pltpu.VMEM((GRP, H), jnp.float32),
            pltpu.SemaphoreType.DMA,
            pltpu.SemaphoreType.DMA,
            pltpu.SemaphoreType.DMA,
            pltpu.SemaphoreType.DMA,
        ],
    )
    def k(ps_hbm, pr_hbm, s_hbm, r_hbm, out_hbm,
          si_v, ri_v, bs0, br0, bs1, br1, semg0, semg1, semo0, semo1):
        wid = lax.axis_index("s") * NC + lax.axis_index("c")
        pltpu.sync_copy(s_hbm.at[wid], si_v)
        pltpu.sync_copy(r_hbm.at[wid], ri_v)
        row0 = wid * NGRP

        def fire(g, bs, br, semg):
            for j in range(NB):
                pltpu.async_copy(ps_hbm.at[si_v.at[g * NB + j]],
                                 bs.at[pl.ds(j * SUB, SUB)], semg)
                pltpu.async_copy(pr_hbm.at[ri_v.at[g * NB + j]],
                                 br.at[pl.ds(j * SUB, SUB)], semg)

        def out_slice(g):
            return out_hbm.at[pl.ds((row0 + g) * GRP, GRP)]

        def finish(g, bs, br, semg, semo):
            for j in range(2 * NB):
                pltpu.make_async_copy(ps_hbm.at[si_v.at[0]],
                                      bs.at[pl.ds(0, SUB)], semg).wait()

            def addb(e, _):
                for cc in range(H // 16):
                    sl = pl.ds(cc * 16, 16)
                    plsc.addupdate(bs.at[e, sl], br[e, sl])
                return 0

            lax.fori_loop(0, GRP, addb, 0)
            pltpu.async_copy(bs, out_slice(g), semo)

        def wait_out(g, bs, semo):
            pltpu.make_async_copy(bs, out_slice(g), semo).wait()

        fire(0, bs0, br0, semg0)

        def body(k2, _):
            g0 = 2 * k2

            @pl.when(k2 > 0)
            def _():
                wait_out(g0 - 1, bs1, semo1)

            @pl.when(g0 + 1 < NGRP)
            def _():
                fire(g0 + 1, bs1, br1, semg1)

            finish(g0, bs0, br0, semg0, semo0)

            @pl.when(g0 + 2 < NGRP)
            def _():
                wait_out(g0, bs0, semo0)
                fire(g0 + 2, bs0, br0, semg0)

            @pl.when(g0 + 1 < NGRP)
            def _():
                finish(g0 + 1, bs1, br1, semg1, semo1)

            return 0

        lax.fori_loop(0, (NGRP + 1) // 2, body, 0)
        if NGRP % 2 == 0:
            wait_out(NGRP - 2, bs0, semo0)
            wait_out(NGRP - 1, bs1, semo1)
        else:
            wait_out(NGRP - 1, bs0, semo0)

    return k(ps, pr, sidx, ridx)


def _scatter_add(newe, ridx):
    mesh = plsc.VectorSubcoreMesh(core_axis_name="c", subcore_axis_name="s")

    @functools.partial(
        pl.kernel,
        out_type=jax.ShapeDtypeStruct((NC * N_NODES, H), jnp.float32),
        mesh=mesh,
        scratch_types=[
            pltpu.VMEM((NGRP_S, SUB_S), jnp.int32),
            pltpu.VMEM((SUB_S, H), jnp.float32),
            pltpu.VMEM((SUB_S, H), jnp.float32),
            pltpu.VMEM_SHARED((N_NODES, H), jnp.float32),
            pltpu.SemaphoreType.DMA,
            pltpu.SemaphoreType.DMA,
        ],
    )
    def k(e_hbm, r_hbm, out_hbm, ri_v, d0, d1, acc_sh, sem0, sem1):
        cid = lax.axis_index("c")
        sid = lax.axis_index("s")
        wid = sid * NC + cid
        pltpu.sync_copy(r_hbm.at[wid], ri_v)

        def zb(e, _):
            for cc in range(H // 16):
                d0[e, pl.ds(cc * 16, 16)] = jnp.zeros((16,), jnp.float32)
            return 0

        lax.fori_loop(0, SUB_S, zb, 0)
        for j in range(8):
            ch = sid + NS * j

            @pl.when(ch < NROWCH)
            def _():
                pltpu.sync_copy(d0, acc_sh.at[pl.ds(ch * SUB_S, SUB_S)])

        plsc.subcore_barrier()

        ebase = wid * EPW

        def fire(g, d, sem):
            pltpu.async_copy(e_hbm.at[pl.ds(ebase + g * SUB_S, SUB_S)], d, sem)

        def finish(g, d, sem):
            pltpu.make_async_copy(e_hbm.at[pl.ds(0, SUB_S)], d, sem).wait()
            pltpu.sync_copy(d, acc_sh.at[ri_v.at[g]], add=True)

        fire(0, d0, sem0)

        def body(k2, _):
            g0 = 2 * k2

            @pl.when(g0 + 1 < NGRP_S)
            def _():
                fire(g0 + 1, d1, sem1)

            finish(g0, d0, sem0)

            @pl.when(g0 + 2 < NGRP_S)
            def _():
                fire(g0 + 2, d0, sem0)

            @pl.when(g0 + 1 < NGRP_S)
            def _():
                finish(g0 + 1, d1, sem1)

            return 0

        lax.fori_loop(0, (NGRP_S + 1) // 2, body, 0)
        plsc.subcore_barrier()

        for j in range(8):
            ch = sid + NS * j

            @pl.when(ch < NROWCH)
            def _():
                pltpu.sync_copy(acc_sh.at[pl.ds(ch * SUB_S, SUB_S)],
                                out_hbm.at[pl.ds(cid * N_NODES + ch * SUB_S, SUB_S)])

    return k(newe, ridx)


def kernel(vertex_features, edge_features, edge_index, eW1, eb1, eW2, eb2,
           vW1, vb1, vW2, vb2):
    senders = edge_index[0].astype(jnp.int32)
    receivers = edge_index[1].astype(jnp.int32)
    sidx = senders.reshape(NW, NGRP * NB, SUB)
    ridx = receivers.reshape(NW, NGRP * NB, SUB)
    ridx_s = receivers.reshape(NW, NGRP_S, SUB_S)

    ws, wr, we = eW1[:H], eW1[H:2 * H], eW1[2 * H:]
    ps, pr = _premix(vertex_features, ws, wr)
    g = _gather_add(ps, pr, sidx, ridx)
    e1 = _edgein(edge_features, we, eb1)
    new_edge = _combine(g, e1, eW2, eb2)
    partials = _scatter_add(new_edge, ridx_s)
    partials = partials.reshape(NC, N_NODES, H)
    new_vertex = _vertex_mlp(vertex_features, partials, vW1[:H], vW1[H:],
                             vb1, vW2, vb2)
    return (new_vertex, new_edge)


# speedup vs baseline: 1.2904x; 1.2904x over previous
import functools

import jax
import jax.numpy as jnp
from jax import lax
from jax.experimental import pallas as pl
from jax.experimental.pallas import tpu as pltpu
from jax.experimental.pallas import tpu_sc as plsc

N_NODES = 10000
N_EDGES = 320000
H = 128

NC = 2
NS = 16
NW = NC * NS
EPW = N_EDGES // NW

SUB = 80
NB = 1
GRP = SUB * NB
NGRP = EPW // GRP

SUB_S = 80
NGRP_S = EPW // SUB_S
NROWCH = N_NODES // SUB_S


def _premix_body(v_ref, ws_ref, wr_ref, ps_ref, pr_ref):
    v = v_ref[...]
    ps_ref[...] = jnp.dot(v, ws_ref[...], preferred_element_type=jnp.float32)
    pr_ref[...] = jnp.dot(v, wr_ref[...], preferred_element_type=jnp.float32)


def _premix(v, ws, wr):
    return pl.pallas_call(
        _premix_body,
        out_shape=(
            jax.ShapeDtypeStruct((N_NODES, H), jnp.float32),
            jax.ShapeDtypeStruct((N_NODES, H), jnp.float32),
        ),
    )(v, ws, wr)


def _combine_body(g_ref, e_ref, we_ref, b1_ref, w2_ref, b2_ref, o_ref):
    pre = (g_ref[...]
           + jnp.dot(e_ref[...], we_ref[...], preferred_element_type=jnp.float32)
           + b1_ref[...])
    h = jnp.maximum(pre, 0.0)
    o_ref[...] = jnp.dot(h, w2_ref[...], preferred_element_type=jnp.float32) + b2_ref[...]


def _combine(g, e, we, b1, w2, b2):
    bm = 512
    return pl.pallas_call(
        _combine_body,
        grid=(N_EDGES // bm,),
        in_specs=[
            pl.BlockSpec((bm, H), lambda i: (i, 0)),
            pl.BlockSpec((bm, H), lambda i: (i, 0)),
            pl.BlockSpec((H, H), lambda i: (0, 0)),
            pl.BlockSpec((1, H), lambda i: (0, 0)),
            pl.BlockSpec((H, H), lambda i: (0, 0)),
            pl.BlockSpec((1, H), lambda i: (0, 0)),
        ],
        out_specs=pl.BlockSpec((bm, H), lambda i: (i, 0)),
        out_shape=jax.ShapeDtypeStruct((N_EDGES, H), jnp.float32),
    )(g, e, we, b1.reshape(1, H), w2, b2.reshape(1, H))


def _vertex_body(v_ref, p_ref, wv_ref, wa_ref, b1_ref, w2_ref, b2_ref, o_ref):
    aggr = p_ref[0] + p_ref[1]
    pre = (jnp.dot(v_ref[...], wv_ref[...], preferred_element_type=jnp.float32)
           + jnp.dot(aggr, wa_ref[...], preferred_element_type=jnp.float32)
           + b1_ref[...])
    h = jnp.maximum(pre, 0.0)
    o_ref[...] = jnp.dot(h, w2_ref[...], preferred_element_type=jnp.float32) + b2_ref[...]


def _vertex_mlp(v, partials, wv, wa, b1, w2, b2):
    return pl.pallas_call(
        _vertex_body,
        out_shape=jax.ShapeDtypeStruct((N_NODES, H), jnp.float32),
    )(v, partials, wv, wa, b1.reshape(1, H), w2, b2.reshape(1, H))


def _gather_add(ps, pr, sidx, ridx):
    mesh = plsc.VectorSubcoreMesh(core_axis_name="c", subcore_axis_name="s")

    @functools.partial(
        pl.kernel,
        out_type=jax.ShapeDtypeStruct((N_EDGES, H), jnp.float32),
        mesh=mesh,
        scratch_types=[
            pltpu.VMEM((NGRP * NB, SUB), jnp.int32),
            pltpu.VMEM((NGRP * NB, SUB), jnp.int32),
            pltpu.VMEM((GRP, H), jnp.float32),
            pltpu.VMEM((GRP, H), jnp.float32),
            pltpu.VMEM((GRP, H), jnp.float32),
            pltpu.VMEM((GRP, H), jnp.float32),
            pltpu.SemaphoreType.DMA,
            pltpu.SemaphoreType.DMA,
            pltpu.SemaphoreType.DMA,
            pltpu.SemaphoreType.DMA,
        ],
    )
    def k(ps_hbm, pr_hbm, s_hbm, r_hbm, out_hbm,
          si_v, ri_v, bs0, br0, bs1, br1, semg0, semg1, semo0, semo1):
        wid = lax.axis_index("s") * NC + lax.axis_index("c")
        pltpu.sync_copy(s_hbm.at[wid], si_v)
        pltpu.sync_copy(r_hbm.at[wid], ri_v)
        row0 = wid * NGRP

        def fire(g, bs, br, semg):
            for j in range(NB):
                pltpu.async_copy(ps_hbm.at[si_v.at[g * NB + j]],
                                 bs.at[pl.ds(j * SUB, SUB)], semg)
                pltpu.async_copy(pr_hbm.at[ri_v.at[g * NB + j]],
                                 br.at[pl.ds(j * SUB, SUB)], semg)

        def out_slice(g):
            return out_hbm.at[pl.ds((row0 + g) * GRP, GRP)]

        def finish(g, bs, br, semg, semo):
            for j in range(2 * NB):
                pltpu.make_async_copy(ps_hbm.at[si_v.at[0]],
                                      bs.at[pl.ds(0, SUB)], semg).wait()

            def addb(e, _):
                for cc in range(H // 16):
                    sl = pl.ds(cc * 16, 16)
                    plsc.addupdate(bs.at[e, sl], br[e, sl])
                return 0

            lax.fori_loop(0, GRP, addb, 0)
            pltpu.async_copy(bs, out_slice(g), semo)

        def wait_out(g, bs, semo):
            pltpu.make_async_copy(bs, out_slice(g), semo).wait()

        fire(0, bs0, br0, semg0)

        def body(k2, _):
            g0 = 2 * k2

            @pl.when(k2 > 0)
            def _():
                wait_out(g0 - 1, bs1, semo1)

            @pl.when(g0 + 1 < NGRP)
            def _():
                fire(g0 + 1, bs1, br1, semg1)

            finish(g0, bs0, br0, semg0, semo0)

            @pl.when(g0 + 2 < NGRP)
            def _():
                wait_out(g0, bs0, semo0)
                fire(g0 + 2, bs0, br0, semg0)

            @pl.when(g0 + 1 < NGRP)
            def _():
                finish(g0 + 1, bs1, br1, semg1, semo1)

            return 0

        lax.fori_loop(0, (NGRP + 1) // 2, body, 0)
        if NGRP % 2 == 0:
            wait_out(NGRP - 2, bs0, semo0)
            wait_out(NGRP - 1, bs1, semo1)
        else:
            wait_out(NGRP - 1, bs0, semo0)

    return k(ps, pr, sidx, ridx)


def _scatter_add(newe, ridx):
    mesh = plsc.VectorSubcoreMesh(core_axis_name="c", subcore_axis_name="s")

    @functools.partial(
        pl.kernel,
        out_type=jax.ShapeDtypeStruct((NC * N_NODES, H), jnp.float32),
        mesh=mesh,
        scratch_types=[
            pltpu.VMEM((NGRP_S, SUB_S), jnp.int32),
            pltpu.VMEM((SUB_S, H), jnp.float32),
            pltpu.VMEM((SUB_S, H), jnp.float32),
            pltpu.VMEM_SHARED((N_NODES, H), jnp.float32),
            pltpu.SemaphoreType.DMA,
            pltpu.SemaphoreType.DMA,
        ],
    )
    def k(e_hbm, r_hbm, out_hbm, ri_v, d0, d1, acc_sh, sem0, sem1):
        cid = lax.axis_index("c")
        sid = lax.axis_index("s")
        wid = sid * NC + cid
        pltpu.sync_copy(r_hbm.at[wid], ri_v)

        def zb(e, _):
            for cc in range(H // 16):
                d0[e, pl.ds(cc * 16, 16)] = jnp.zeros((16,), jnp.float32)
            return 0

        lax.fori_loop(0, SUB_S, zb, 0)
        for j in range(8):
            ch = sid + NS * j

            @pl.when(ch < NROWCH)
            def _():
                pltpu.sync_copy(d0, acc_sh.at[pl.ds(ch * SUB_S, SUB_S)])

        plsc.subcore_barrier()

        ebase = wid * EPW

        def fire(g, d, sem):
            pltpu.async_copy(e_hbm.at[pl.ds(ebase + g * SUB_S, SUB_S)], d, sem)

        def finish(g, d, sem):
            pltpu.make_async_copy(e_hbm.at[pl.ds(0, SUB_S)], d, sem).wait()
            pltpu.sync_copy(d, acc_sh.at[ri_v.at[g]], add=True)

        fire(0, d0, sem0)

        def body(k2, _):
            g0 = 2 * k2

            @pl.when(g0 + 1 < NGRP_S)
            def _():
                fire(g0 + 1, d1, sem1)

            finish(g0, d0, sem0)

            @pl.when(g0 + 2 < NGRP_S)
            def _():
                fire(g0 + 2, d0, sem0)

            @pl.when(g0 + 1 < NGRP_S)
            def _():
                finish(g0 + 1, d1, sem1)

            return 0

        lax.fori_loop(0, (NGRP_S + 1) // 2, body, 0)
        plsc.subcore_barrier()

        for j in range(8):
            ch = sid + NS * j

            @pl.when(ch < NROWCH)
            def _():
                pltpu.sync_copy(acc_sh.at[pl.ds(ch * SUB_S, SUB_S)],
                                out_hbm.at[pl.ds(cid * N_NODES + ch * SUB_S, SUB_S)])

    return k(newe, ridx)


def kernel(vertex_features, edge_features, edge_index, eW1, eb1, eW2, eb2,
           vW1, vb1, vW2, vb2):
    senders = edge_index[0].astype(jnp.int32)
    receivers = edge_index[1].astype(jnp.int32)
    sidx = senders.reshape(NW, NGRP * NB, SUB)
    ridx = receivers.reshape(NW, NGRP * NB, SUB)
    ridx_s = receivers.reshape(NW, NGRP_S, SUB_S)

    ws, wr, we = eW1[:H], eW1[H:2 * H], eW1[2 * H:]
    ps, pr = _premix(vertex_features, ws, wr)
    g = _gather_add(ps, pr, sidx, ridx)
    new_edge = _combine(g, edge_features, we, eb1, eW2, eb2)
    partials = _scatter_add(new_edge, ridx_s)
    partials = partials.reshape(NC, N_NODES, H)
    new_vertex = _vertex_mlp(vertex_features, partials, vW1[:H], vW1[H:],
                             vb1, vW2, vb2)
    return (new_vertex, new_edge)


# speedup vs baseline: 2.0399x; 1.5808x over previous
import functools

import jax
import jax.numpy as jnp
from jax import lax
from jax.experimental import pallas as pl
from jax.experimental.pallas import tpu as pltpu
from jax.experimental.pallas import tpu_sc as plsc

N_NODES = 10000
N_EDGES = 320000
H = 128

NC = 2
NS = 16
NW = NC * NS
EPW = N_EDGES // NW

SUB = 80
NB = 1
GRP = SUB * NB
NGRP = EPW // GRP

SUB_S = 80
NGRP_S = EPW // SUB_S
NROWCH = N_NODES // SUB_S


def _premix_body(v_ref, ws_ref, wr_ref, ps_ref, pr_ref):
    v = v_ref[...]
    ps_ref[...] = jnp.dot(v, ws_ref[...], preferred_element_type=jnp.float32)
    pr_ref[...] = jnp.dot(v, wr_ref[...], preferred_element_type=jnp.float32)


def _premix(v, ws, wr):
    return pl.pallas_call(
        _premix_body,
        out_shape=(
            jax.ShapeDtypeStruct((N_NODES, H), jnp.float32),
            jax.ShapeDtypeStruct((N_NODES, H), jnp.float32),
        ),
    )(v, ws, wr)


def _combine_body(g_ref, e_ref, we_ref, b1_ref, w2_ref, b2_ref, o_ref):
    pre = (g_ref[...]
           + jnp.dot(e_ref[...], we_ref[...], preferred_element_type=jnp.float32)
           + b1_ref[...])
    h = jnp.maximum(pre, 0.0)
    o_ref[...] = jnp.dot(h, w2_ref[...], preferred_element_type=jnp.float32) + b2_ref[...]


def _combine(g, e, we, b1, w2, b2):
    bm = 3200
    return pl.pallas_call(
        _combine_body,
        grid=(N_EDGES // bm,),
        in_specs=[
            pl.BlockSpec((bm, H), lambda i: (i, 0)),
            pl.BlockSpec((bm, H), lambda i: (i, 0)),
            pl.BlockSpec((H, H), lambda i: (0, 0)),
            pl.BlockSpec((1, H), lambda i: (0, 0)),
            pl.BlockSpec((H, H), lambda i: (0, 0)),
            pl.BlockSpec((1, H), lambda i: (0, 0)),
        ],
        out_specs=pl.BlockSpec((bm, H), lambda i: (i, 0)),
        out_shape=jax.ShapeDtypeStruct((N_EDGES, H), jnp.float32),
    )(g, e, we, b1.reshape(1, H), w2, b2.reshape(1, H))


def _vertex_body(v_ref, p_ref, wv_ref, wa_ref, b1_ref, w2_ref, b2_ref, o_ref):
    aggr = p_ref[0] + p_ref[1]
    pre = (jnp.dot(v_ref[...], wv_ref[...], preferred_element_type=jnp.float32)
           + jnp.dot(aggr, wa_ref[...], preferred_element_type=jnp.float32)
           + b1_ref[...])
    h = jnp.maximum(pre, 0.0)
    o_ref[...] = jnp.dot(h, w2_ref[...], preferred_element_type=jnp.float32) + b2_ref[...]


def _vertex_mlp(v, partials, wv, wa, b1, w2, b2):
    return pl.pallas_call(
        _vertex_body,
        out_shape=jax.ShapeDtypeStruct((N_NODES, H), jnp.float32),
    )(v, partials, wv, wa, b1.reshape(1, H), w2, b2.reshape(1, H))


def _gather_add(ps, pr, sidx, ridx):
    mesh = plsc.VectorSubcoreMesh(core_axis_name="c", subcore_axis_name="s")

    @functools.partial(
        pl.kernel,
        out_type=jax.ShapeDtypeStruct((N_EDGES, H), jnp.float32),
        mesh=mesh,
        scratch_types=[
            pltpu.VMEM((NGRP * NB, SUB), jnp.int32),
            pltpu.VMEM((NGRP * NB, SUB), jnp.int32),
            pltpu.VMEM((GRP, H), jnp.float32),
            pltpu.VMEM((GRP, H), jnp.float32),
            pltpu.VMEM((GRP, H), jnp.float32),
            pltpu.VMEM((GRP, H), jnp.float32),
            pltpu.SemaphoreType.DMA,
            pltpu.SemaphoreType.DMA,
            pltpu.SemaphoreType.DMA,
            pltpu.SemaphoreType.DMA,
        ],
    )
    def k(ps_hbm, pr_hbm, s_hbm, r_hbm, out_hbm,
          si_v, ri_v, bs0, br0, bs1, br1, semg0, semg1, semo0, semo1):
        wid = lax.axis_index("s") * NC + lax.axis_index("c")
        pltpu.sync_copy(s_hbm.at[wid], si_v)
        pltpu.sync_copy(r_hbm.at[wid], ri_v)
        row0 = wid * NGRP

        def fire(g, bs, br, semg):
            for j in range(NB):
                pltpu.async_copy(ps_hbm.at[si_v.at[g * NB + j]],
                                 bs.at[pl.ds(j * SUB, SUB)], semg)
                pltpu.async_copy(pr_hbm.at[ri_v.at[g * NB + j]],
                                 br.at[pl.ds(j * SUB, SUB)], semg)

        def out_slice(g):
            return out_hbm.at[pl.ds((row0 + g) * GRP, GRP)]

        def finish(g, bs, br, semg, semo):
            for j in range(2 * NB):
                pltpu.make_async_copy(ps_hbm.at[si_v.at[0]],
                                      bs.at[pl.ds(0, SUB)], semg).wait()

            def addb(e, _):
                for cc in range(H // 16):
                    sl = pl.ds(cc * 16, 16)
                    plsc.addupdate(bs.at[e, sl], br[e, sl])
                return 0

            lax.fori_loop(0, GRP, addb, 0)
            pltpu.async_copy(bs, out_slice(g), semo)

        def wait_out(g, bs, semo):
            pltpu.make_async_copy(bs, out_slice(g), semo).wait()

        fire(0, bs0, br0, semg0)

        def body(k2, _):
            g0 = 2 * k2

            @pl.when(k2 > 0)
            def _():
                wait_out(g0 - 1, bs1, semo1)

            @pl.when(g0 + 1 < NGRP)
            def _():
                fire(g0 + 1, bs1, br1, semg1)

            finish(g0, bs0, br0, semg0, semo0)

            @pl.when(g0 + 2 < NGRP)
            def _():
                wait_out(g0, bs0, semo0)
                fire(g0 + 2, bs0, br0, semg0)

            @pl.when(g0 + 1 < NGRP)
            def _():
                finish(g0 + 1, bs1, br1, semg1, semo1)

            return 0

        lax.fori_loop(0, (NGRP + 1) // 2, body, 0)
        if NGRP % 2 == 0:
            wait_out(NGRP - 2, bs0, semo0)
            wait_out(NGRP - 1, bs1, semo1)
        else:
            wait_out(NGRP - 1, bs0, semo0)

    return k(ps, pr, sidx, ridx)


def _scatter_add(newe, ridx):
    mesh = plsc.VectorSubcoreMesh(core_axis_name="c", subcore_axis_name="s")

    @functools.partial(
        pl.kernel,
        out_type=jax.ShapeDtypeStruct((NC * N_NODES, H), jnp.float32),
        mesh=mesh,
        scratch_types=[
            pltpu.VMEM((NGRP_S, SUB_S), jnp.int32),
            pltpu.VMEM((SUB_S, H), jnp.float32),
            pltpu.VMEM((SUB_S, H), jnp.float32),
            pltpu.VMEM_SHARED((N_NODES, H), jnp.float32),
            pltpu.SemaphoreType.DMA,
            pltpu.SemaphoreType.DMA,
        ],
    )
    def k(e_hbm, r_hbm, out_hbm, ri_v, d0, d1, acc_sh, sem0, sem1):
        cid = lax.axis_index("c")
        sid = lax.axis_index("s")
        wid = sid * NC + cid
        pltpu.sync_copy(r_hbm.at[wid], ri_v)

        def zb(e, _):
            for cc in range(H // 16):
                d0[e, pl.ds(cc * 16, 16)] = jnp.zeros((16,), jnp.float32)
            return 0

        lax.fori_loop(0, SUB_S, zb, 0)
        for j in range(8):
            ch = sid + NS * j

            @pl.when(ch < NROWCH)
            def _():
                pltpu.sync_copy(d0, acc_sh.at[pl.ds(ch * SUB_S, SUB_S)])

        plsc.subcore_barrier()

        ebase = wid * EPW

        def fire(g, d, sem):
            pltpu.async_copy(e_hbm.at[pl.ds(ebase + g * SUB_S, SUB_S)], d, sem)

        def finish(g, d, sem):
            pltpu.make_async_copy(e_hbm.at[pl.ds(0, SUB_S)], d, sem).wait()
            pltpu.sync_copy(d, acc_sh.at[ri_v.at[g]], add=True)

        fire(0, d0, sem0)

        def body(k2, _):
            g0 = 2 * k2

            @pl.when(g0 + 1 < NGRP_S)
            def _():
                fire(g0 + 1, d1, sem1)

            finish(g0, d0, sem0)

            @pl.when(g0 + 2 < NGRP_S)
            def _():
                fire(g0 + 2, d0, sem0)

            @pl.when(g0 + 1 < NGRP_S)
            def _():
                finish(g0 + 1, d1, sem1)

            return 0

        lax.fori_loop(0, (NGRP_S + 1) // 2, body, 0)
        plsc.subcore_barrier()

        for j in range(8):
            ch = sid + NS * j

            @pl.when(ch < NROWCH)
            def _():
                pltpu.sync_copy(acc_sh.at[pl.ds(ch * SUB_S, SUB_S)],
                                out_hbm.at[pl.ds(cid * N_NODES + ch * SUB_S, SUB_S)])

    return k(newe, ridx)


def kernel(vertex_features, edge_features, edge_index, eW1, eb1, eW2, eb2,
           vW1, vb1, vW2, vb2):
    senders = edge_index[0].astype(jnp.int32)
    receivers = edge_index[1].astype(jnp.int32)
    sidx = senders.reshape(NW, NGRP * NB, SUB)
    ridx = receivers.reshape(NW, NGRP * NB, SUB)
    ridx_s = receivers.reshape(NW, NGRP_S, SUB_S)

    ws, wr, we = eW1[:H], eW1[H:2 * H], eW1[2 * H:]
    ps, pr = _premix(vertex_features, ws, wr)
    g = _gather_add(ps, pr, sidx, ridx)
    new_edge = _combine(g, edge_features, we, eb1, eW2, eb2)
    partials = _scatter_add(new_edge, ridx_s)
    partials = partials.reshape(NC, N_NODES, H)
    new_vertex = _vertex_mlp(vertex_features, partials, vW1[:H], vW1[H:],
                             vb1, vW2, vb2)
    return (new_vertex, new_edge)


# speedup vs baseline: 2.1218x; 1.0402x over previous
import functools

import jax
import jax.numpy as jnp
from jax import lax
from jax.experimental import pallas as pl
from jax.experimental.pallas import tpu as pltpu
from jax.experimental.pallas import tpu_sc as plsc

N_NODES = 10000
N_EDGES = 320000
H = 128

NC = 2
NS = 16
NW = NC * NS
EPW = N_EDGES // NW

SUB = 80
NB = 1
GRP = SUB * NB
NGRP = EPW // GRP

SUB_S = 80
NGRP_S = EPW // SUB_S
NROWCH = N_NODES // SUB_S


def _premix_body(v_ref, ws_ref, wr_ref, ps_ref, pr_ref):
    v = v_ref[...]
    ps_ref[...] = jnp.dot(v, ws_ref[...], preferred_element_type=jnp.float32)
    pr_ref[...] = jnp.dot(v, wr_ref[...], preferred_element_type=jnp.float32)


def _premix(v, ws, wr):
    return pl.pallas_call(
        _premix_body,
        out_shape=(
            jax.ShapeDtypeStruct((N_NODES, H), jnp.float32),
            jax.ShapeDtypeStruct((N_NODES, H), jnp.float32),
        ),
    )(v, ws, wr)


def _combine_body(g_ref, e_ref, we_ref, b1_ref, w2_ref, b2_ref, o_ref):
    pre = (g_ref[...]
           + jnp.dot(e_ref[...], we_ref[...], preferred_element_type=jnp.float32)
           + b1_ref[...])
    h = jnp.maximum(pre, 0.0)
    o_ref[...] = jnp.dot(h, w2_ref[...], preferred_element_type=jnp.float32) + b2_ref[...]


def _combine(g, e, we, b1, w2, b2):
    bm = 6400
    return pl.pallas_call(
        _combine_body,
        grid=(N_EDGES // bm,),
        in_specs=[
            pl.BlockSpec((bm, H), lambda i: (i, 0)),
            pl.BlockSpec((bm, H), lambda i: (i, 0)),
            pl.BlockSpec((H, H), lambda i: (0, 0)),
            pl.BlockSpec((1, H), lambda i: (0, 0)),
            pl.BlockSpec((H, H), lambda i: (0, 0)),
            pl.BlockSpec((1, H), lambda i: (0, 0)),
        ],
        out_specs=pl.BlockSpec((bm, H), lambda i: (i, 0)),
        out_shape=jax.ShapeDtypeStruct((N_EDGES, H), jnp.float32),
    )(g, e, we, b1.reshape(1, H), w2, b2.reshape(1, H))


def _vertex_body(v_ref, p_ref, wv_ref, wa_ref, b1_ref, w2_ref, b2_ref, o_ref):
    aggr = p_ref[0] + p_ref[1]
    pre = (jnp.dot(v_ref[...], wv_ref[...], preferred_element_type=jnp.float32)
           + jnp.dot(aggr, wa_ref[...], preferred_element_type=jnp.float32)
           + b1_ref[...])
    h = jnp.maximum(pre, 0.0)
    o_ref[...] = jnp.dot(h, w2_ref[...], preferred_element_type=jnp.float32) + b2_ref[...]


def _vertex_mlp(v, partials, wv, wa, b1, w2, b2):
    return pl.pallas_call(
        _vertex_body,
        out_shape=jax.ShapeDtypeStruct((N_NODES, H), jnp.float32),
    )(v, partials, wv, wa, b1.reshape(1, H), w2, b2.reshape(1, H))


def _gather_add(ps, pr, sidx, ridx):
    mesh = plsc.VectorSubcoreMesh(core_axis_name="c", subcore_axis_name="s")

    @functools.partial(
        pl.kernel,
        out_type=jax.ShapeDtypeStruct((N_EDGES, H), jnp.float32),
        mesh=mesh,
        scratch_types=[
            pltpu.VMEM((NGRP * NB, SUB), jnp.int32),
            pltpu.VMEM((NGRP * NB, SUB), jnp.int32),
            pltpu.VMEM((GRP, H), jnp.float32),
            pltpu.VMEM((GRP, H), jnp.float32),
            pltpu.VMEM((GRP, H), jnp.float32),
            pltpu.VMEM((GRP, H), jnp.float32),
            pltpu.SemaphoreType.DMA,
            pltpu.SemaphoreType.DMA,
            pltpu.SemaphoreType.DMA,
            pltpu.SemaphoreType.DMA,
        ],
    )
    def k(ps_hbm, pr_hbm, s_hbm, r_hbm, out_hbm,
          si_v, ri_v, bs0, br0, bs1, br1, semg0, semg1, semo0, semo1):
        wid = lax.axis_index("s") * NC + lax.axis_index("c")
        pltpu.sync_copy(s_hbm.at[wid], si_v)
        pltpu.sync_copy(r_hbm.at[wid], ri_v)
        row0 = wid * NGRP

        def fire(g, bs, br, semg):
            for j in range(NB):
                pltpu.async_copy(ps_hbm.at[si_v.at[g * NB + j]],
                                 bs.at[pl.ds(j * SUB, SUB)], semg)
                pltpu.async_copy(pr_hbm.at[ri_v.at[g * NB + j]],
                                 br.at[pl.ds(j * SUB, SUB)], semg)

        def out_slice(g):
            return out_hbm.at[pl.ds((row0 + g) * GRP, GRP)]

        def finish(g, bs, br, semg, semo):
            for j in range(2 * NB):
                pltpu.make_async_copy(ps_hbm.at[si_v.at[0]],
                                      bs.at[pl.ds(0, SUB)], semg).wait()

            def addb(e, _):
                for cc in range(H // 16):
                    sl = pl.ds(cc * 16, 16)
                    plsc.addupdate(bs.at[e, sl], br[e, sl])
                return 0

            lax.fori_loop(0, GRP, addb, 0)
            pltpu.async_copy(bs, out_slice(g), semo)

        def wait_out(g, bs, semo):
            pltpu.make_async_copy(bs, out_slice(g), semo).wait()

        fire(0, bs0, br0, semg0)

        def body(k2, _):
            g0 = 2 * k2

            @pl.when(k2 > 0)
            def _():
                wait_out(g0 - 1, bs1, semo1)

            @pl.when(g0 + 1 < NGRP)
            def _():
                fire(g0 + 1, bs1, br1, semg1)

            finish(g0, bs0, br0, semg0, semo0)

            @pl.when(g0 + 2 < NGRP)
            def _():
                wait_out(g0, bs0, semo0)
                fire(g0 + 2, bs0, br0, semg0)

            @pl.when(g0 + 1 < NGRP)
            def _():
                finish(g0 + 1, bs1, br1, semg1, semo1)

            return 0

        lax.fori_loop(0, (NGRP + 1) // 2, body, 0)
        if NGRP % 2 == 0:
            wait_out(NGRP - 2, bs0, semo0)
            wait_out(NGRP - 1, bs1, semo1)
        else:
            wait_out(NGRP - 1, bs0, semo0)

    return k(ps, pr, sidx, ridx)


def _scatter_add(newe, ridx):
    mesh = plsc.VectorSubcoreMesh(core_axis_name="c", subcore_axis_name="s")

    @functools.partial(
        pl.kernel,
        out_type=jax.ShapeDtypeStruct((NC * N_NODES, H), jnp.float32),
        mesh=mesh,
        scratch_types=[
            pltpu.VMEM((NGRP_S, SUB_S), jnp.int32),
            pltpu.VMEM((SUB_S, H), jnp.float32),
            pltpu.VMEM((SUB_S, H), jnp.float32),
            pltpu.VMEM_SHARED((N_NODES, H), jnp.float32),
            pltpu.SemaphoreType.DMA,
            pltpu.SemaphoreType.DMA,
        ],
    )
    def k(e_hbm, r_hbm, out_hbm, ri_v, d0, d1, acc_sh, sem0, sem1):
        cid = lax.axis_index("c")
        sid = lax.axis_index("s")
        wid = sid * NC + cid
        pltpu.sync_copy(r_hbm.at[wid], ri_v)

        def zb(e, _):
            for cc in range(H // 16):
                d0[e, pl.ds(cc * 16, 16)] = jnp.zeros((16,), jnp.float32)
            return 0

        lax.fori_loop(0, SUB_S, zb, 0)
        for j in range(8):
            ch = sid + NS * j

            @pl.when(ch < NROWCH)
            def _():
                pltpu.sync_copy(d0, acc_sh.at[pl.ds(ch * SUB_S, SUB_S)])

        plsc.subcore_barrier()

        ebase = wid * EPW

        def fire(g, d, sem):
            pltpu.async_copy(e_hbm.at[pl.ds(ebase + g * SUB_S, SUB_S)], d, sem)

        def finish(g, d, sem):
            pltpu.make_async_copy(e_hbm.at[pl.ds(0, SUB_S)], d, sem).wait()
            pltpu.sync_copy(d, acc_sh.at[ri_v.at[g]], add=True)

        fire(0, d0, sem0)

        def body(k2, _):
            g0 = 2 * k2

            @pl.when(g0 + 1 < NGRP_S)
            def _():
                fire(g0 + 1, d1, sem1)

            finish(g0, d0, sem0)

            @pl.when(g0 + 2 < NGRP_S)
            def _():
                fire(g0 + 2, d0, sem0)

            @pl.when(g0 + 1 < NGRP_S)
            def _():
                finish(g0 + 1, d1, sem1)

            return 0

        lax.fori_loop(0, (NGRP_S + 1) // 2, body, 0)
        plsc.subcore_barrier()

        for j in range(8):
            ch = sid + NS * j

            @pl.when(ch < NROWCH)
            def _():
                pltpu.sync_copy(acc_sh.at[pl.ds(ch * SUB_S, SUB_S)],
                                out_hbm.at[pl.ds(cid * N_NODES + ch * SUB_S, SUB_S)])

    return k(newe, ridx)


def kernel(vertex_features, edge_features, edge_index, eW1, eb1, eW2, eb2,
           vW1, vb1, vW2, vb2):
    senders = edge_index[0].astype(jnp.int32)
    receivers = edge_index[1].astype(jnp.int32)
    sidx = senders.reshape(NW, NGRP * NB, SUB)
    ridx = receivers.reshape(NW, NGRP * NB, SUB)
    ridx_s = receivers.reshape(NW, NGRP_S, SUB_S)

    ws, wr, we = eW1[:H], eW1[H:2 * H], eW1[2 * H:]
    ps, pr = _premix(vertex_features, ws, wr)
    g = _gather_add(ps, pr, sidx, ridx)
    new_edge = _combine(g, edge_features, we, eb1, eW2, eb2)
    partials = _scatter_add(new_edge, ridx_s)
    partials = partials.reshape(NC, N_NODES, H)
    new_vertex = _vertex_mlp(vertex_features, partials, vW1[:H], vW1[H:],
                             vb1, vW2, vb2)
    return (new_vertex, new_edge)


# speedup vs baseline: 2.1430x; 1.0100x over previous
import functools

import jax
import jax.numpy as jnp
from jax import lax
from jax.experimental import pallas as pl
from jax.experimental.pallas import tpu as pltpu
from jax.experimental.pallas import tpu_sc as plsc

N_NODES = 10000
N_EDGES = 320000
H = 128

NC = 2
NS = 16
NW = NC * NS
EPW = N_EDGES // NW

SUB = 80
NB = 1
GRP = SUB * NB
NGRP = EPW // GRP

SUB_S = 80
NGRP_S = EPW // SUB_S
NROWCH = N_NODES // SUB_S


def _premix_body(v_ref, ws_ref, wr_ref, ps_ref, pr_ref):
    v = v_ref[...]
    ps_ref[...] = jnp.dot(v, ws_ref[...], preferred_element_type=jnp.float32)
    pr_ref[...] = jnp.dot(v, wr_ref[...], preferred_element_type=jnp.float32)


def _premix(v, ws, wr):
    return pl.pallas_call(
        _premix_body,
        out_shape=(
            jax.ShapeDtypeStruct((N_NODES, H), jnp.float32),
            jax.ShapeDtypeStruct((N_NODES, H), jnp.float32),
        ),
    )(v, ws, wr)


def _combine_body(g_ref, e_ref, we_ref, b1_ref, w2_ref, b2_ref, o_ref):
    pre = (g_ref[...]
           + jnp.dot(e_ref[...], we_ref[...], preferred_element_type=jnp.float32)
           + b1_ref[...])
    h = jnp.maximum(pre, 0.0)
    o_ref[...] = jnp.dot(h, w2_ref[...], preferred_element_type=jnp.float32) + b2_ref[...]


def _combine(g, e, we, b1, w2, b2):
    bm = 10000
    return pl.pallas_call(
        _combine_body,
        grid=(N_EDGES // bm,),
        in_specs=[
            pl.BlockSpec((bm, H), lambda i: (i, 0)),
            pl.BlockSpec((bm, H), lambda i: (i, 0)),
            pl.BlockSpec((H, H), lambda i: (0, 0)),
            pl.BlockSpec((1, H), lambda i: (0, 0)),
            pl.BlockSpec((H, H), lambda i: (0, 0)),
            pl.BlockSpec((1, H), lambda i: (0, 0)),
        ],
        out_specs=pl.BlockSpec((bm, H), lambda i: (i, 0)),
        out_shape=jax.ShapeDtypeStruct((N_EDGES, H), jnp.float32),
    )(g, e, we, b1.reshape(1, H), w2, b2.reshape(1, H))


def _vertex_body(v_ref, p_ref, wv_ref, wa_ref, b1_ref, w2_ref, b2_ref, o_ref):
    aggr = p_ref[0] + p_ref[1]
    pre = (jnp.dot(v_ref[...], wv_ref[...], preferred_element_type=jnp.float32)
           + jnp.dot(aggr, wa_ref[...], preferred_element_type=jnp.float32)
           + b1_ref[...])
    h = jnp.maximum(pre, 0.0)
    o_ref[...] = jnp.dot(h, w2_ref[...], preferred_element_type=jnp.float32) + b2_ref[...]


def _vertex_mlp(v, partials, wv, wa, b1, w2, b2):
    return pl.pallas_call(
        _vertex_body,
        out_shape=jax.ShapeDtypeStruct((N_NODES, H), jnp.float32),
    )(v, partials, wv, wa, b1.reshape(1, H), w2, b2.reshape(1, H))


def _gather_add(ps, pr, sidx, ridx):
    mesh = plsc.VectorSubcoreMesh(core_axis_name="c", subcore_axis_name="s")

    @functools.partial(
        pl.kernel,
        out_type=jax.ShapeDtypeStruct((N_EDGES, H), jnp.float32),
        mesh=mesh,
        scratch_types=[
            pltpu.VMEM((NGRP * NB, SUB), jnp.int32),
            pltpu.VMEM((NGRP * NB, SUB), jnp.int32),
            pltpu.VMEM((GRP, H), jnp.float32),
            pltpu.VMEM((GRP, H), jnp.float32),
            pltpu.VMEM((GRP, H), jnp.float32),
            pltpu.VMEM((GRP, H), jnp.float32),
            pltpu.SemaphoreType.DMA,
            pltpu.SemaphoreType.DMA,
            pltpu.SemaphoreType.DMA,
            pltpu.SemaphoreType.DMA,
        ],
    )
    def k(ps_hbm, pr_hbm, s_hbm, r_hbm, out_hbm,
          si_v, ri_v, bs0, br0, bs1, br1, semg0, semg1, semo0, semo1):
        wid = lax.axis_index("s") * NC + lax.axis_index("c")
        pltpu.sync_copy(s_hbm.at[wid], si_v)
        pltpu.sync_copy(r_hbm.at[wid], ri_v)
        row0 = wid * NGRP

        def fire(g, bs, br, semg):
            for j in range(NB):
                pltpu.async_copy(ps_hbm.at[si_v.at[g * NB + j]],
                                 bs.at[pl.ds(j * SUB, SUB)], semg)
                pltpu.async_copy(pr_hbm.at[ri_v.at[g * NB + j]],
                                 br.at[pl.ds(j * SUB, SUB)], semg)

        def out_slice(g):
            return out_hbm.at[pl.ds((row0 + g) * GRP, GRP)]

        def finish(g, bs, br, semg, semo):
            for j in range(2 * NB):
                pltpu.make_async_copy(ps_hbm.at[si_v.at[0]],
                                      bs.at[pl.ds(0, SUB)], semg).wait()

            def addb(e, _):
                for cc in range(H // 16):
                    sl = pl.ds(cc * 16, 16)
                    plsc.addupdate(bs.at[e, sl], br[e, sl])
                return 0

            lax.fori_loop(0, GRP, addb, 0)
            pltpu.async_copy(bs, out_slice(g), semo)

        def wait_out(g, bs, semo):
            pltpu.make_async_copy(bs, out_slice(g), semo).wait()

        fire(0, bs0, br0, semg0)

        def body(k2, _):
            g0 = 2 * k2

            @pl.when(k2 > 0)
            def _():
                wait_out(g0 - 1, bs1, semo1)

            @pl.when(g0 + 1 < NGRP)
            def _():
                fire(g0 + 1, bs1, br1, semg1)

            finish(g0, bs0, br0, semg0, semo0)

            @pl.when(g0 + 2 < NGRP)
            def _():
                wait_out(g0, bs0, semo0)
                fire(g0 + 2, bs0, br0, semg0)

            @pl.when(g0 + 1 < NGRP)
            def _():
                finish(g0 + 1, bs1, br1, semg1, semo1)

            return 0

        lax.fori_loop(0, (NGRP + 1) // 2, body, 0)
        if NGRP % 2 == 0:
            wait_out(NGRP - 2, bs0, semo0)
            wait_out(NGRP - 1, bs1, semo1)
        else:
            wait_out(NGRP - 1, bs0, semo0)

    return k(ps, pr, sidx, ridx)


def _scatter_add(newe, ridx):
    mesh = plsc.VectorSubcoreMesh(core_axis_name="c", subcore_axis_name="s")

    @functools.partial(
        pl.kernel,
        out_type=jax.ShapeDtypeStruct((NC * N_NODES, H), jnp.float32),
        mesh=mesh,
        scratch_types=[
            pltpu.VMEM((NGRP_S, SUB_S), jnp.int32),
            pltpu.VMEM((SUB_S, H), jnp.float32),
            pltpu.VMEM((SUB_S, H), jnp.float32),
            pltpu.VMEM_SHARED((N_NODES, H), jnp.float32),
            pltpu.SemaphoreType.DMA,
            pltpu.SemaphoreType.DMA,
        ],
    )
    def k(e_hbm, r_hbm, out_hbm, ri_v, d0, d1, acc_sh, sem0, sem1):
        cid = lax.axis_index("c")
        sid = lax.axis_index("s")
        wid = sid * NC + cid
        pltpu.sync_copy(r_hbm.at[wid], ri_v)

        def zb(e, _):
            for cc in range(H // 16):
                d0[e, pl.ds(cc * 16, 16)] = jnp.zeros((16,), jnp.float32)
            return 0

        lax.fori_loop(0, SUB_S, zb, 0)
        for j in range(8):
            ch = sid + NS * j

            @pl.when(ch < NROWCH)
            def _():
                pltpu.sync_copy(d0, acc_sh.at[pl.ds(ch * SUB_S, SUB_S)])

        plsc.subcore_barrier()

        ebase = wid * EPW

        def fire(g, d, sem):
            pltpu.async_copy(e_hbm.at[pl.ds(ebase + g * SUB_S, SUB_S)], d, sem)

        def finish(g, d, sem):
            pltpu.make_async_copy(e_hbm.at[pl.ds(0, SUB_S)], d, sem).wait()
            pltpu.sync_copy(d, acc_sh.at[ri_v.at[g]], add=True)

        fire(0, d0, sem0)

        def body(k2, _):
            g0 = 2 * k2

            @pl.when(g0 + 1 < NGRP_S)
            def _():
                fire(g0 + 1, d1, sem1)

            finish(g0, d0, sem0)

            @pl.when(g0 + 2 < NGRP_S)
            def _():
                fire(g0 + 2, d0, sem0)

            @pl.when(g0 + 1 < NGRP_S)
            def _():
                finish(g0 + 1, d1, sem1)

            return 0

        lax.fori_loop(0, (NGRP_S + 1) // 2, body, 0)
        plsc.subcore_barrier()

        for j in range(8):
            ch = sid + NS * j

            @pl.when(ch < NROWCH)
            def _():
                pltpu.sync_copy(acc_sh.at[pl.ds(ch * SUB_S, SUB_S)],
                                out_hbm.at[pl.ds(cid * N_NODES + ch * SUB_S, SUB_S)])

    return k(newe, ridx)


def kernel(vertex_features, edge_features, edge_index, eW1, eb1, eW2, eb2,
           vW1, vb1, vW2, vb2):
    senders = edge_index[0].astype(jnp.int32)
    receivers = edge_index[1].astype(jnp.int32)
    sidx = senders.reshape(NW, NGRP * NB, SUB)
    ridx = receivers.reshape(NW, NGRP * NB, SUB)
    ridx_s = receivers.reshape(NW, NGRP_S, SUB_S)

    ws, wr, we = eW1[:H], eW1[H:2 * H], eW1[2 * H:]
    ps, pr = _premix(vertex_features, ws, wr)
    g = _gather_add(ps, pr, sidx, ridx)
    new_edge = _combine(g, edge_features, we, eb1, eW2, eb2)
    partials = _scatter_add(new_edge, ridx_s)
    partials = partials.reshape(NC, N_NODES, H)
    new_vertex = _vertex_mlp(vertex_features, partials, vW1[:H], vW1[H:],
                             vb1, vW2, vb2)
    return (new_vertex, new_edge)
